# baseline (speedup 1.0000x reference)
import functools

import jax
import jax.numpy as jnp
from jax import lax
from jax.experimental import pallas as pl
from jax.experimental.pallas import tpu as pltpu
from jax.experimental.pallas import tpu_sc as plsc

N = 10000
E = 160000
R = 2
D = 256
DA = 16
DH = D // 2

NC = 2
NS = 16
K = 48
NCH = 216
G = 8
NGRP = NCH // G
EPC = NCH * K
EP = EPC * NS
NROW = 10112
NG = 80
NPA = NROW + NG + 8
GR = NPA - 1
NP = NROW
CL = 16
NPC = NG * 128
WPT = NROW // NS


@functools.lru_cache(maxsize=1)
def _build_sc_aggregate():
  mesh = plsc.VectorSubcoreMesh(core_axis_name="c", subcore_axis_name="s",
                                num_cores=NC, num_subcores=NS)

  @functools.partial(
      pl.kernel,
      out_type=(
          jax.ShapeDtypeStruct((R, NC, NP, DH), jnp.float32),
          jax.ShapeDtypeStruct((R, NG, 128), jnp.float32),
      ),
      mesh=mesh,
      scratch_types=(
          pltpu.VMEM((2, G, K), jnp.int32),
          pltpu.VMEM((2, G, K), jnp.int32),
          pltpu.VMEM((K, DH), jnp.float32),
          pltpu.VMEM((K, DH), jnp.float32),
          pltpu.VMEM_SHARED((NPA, DH), jnp.float32),
          pltpu.SemaphoreType.DMA,
          pltpu.SemaphoreType.DMA,
          pltpu.SemaphoreType.DMA((2,)),
      ),
  )
  def _sc_aggregate(srcA, srcC, locF, locC, feat2e, zeros_d,
                    agg_out, cnt_out,
                    sring, lring, rows_a, rows_b, accum_sh, sem_a, sem_b,
                    isem):
    c = lax.axis_index("c")
    s = lax.axis_index("s")
    zrow = s * WPT

    for r in range(R):
      ngroups = NGRP + jnp.where(c == r, NGRP, 0)

      def stage(g, slot):
          @pl.when(g < NGRP)
          def _():
              pltpu.async_copy(srcA.at[r, c, s, pl.ds(g * G, G)],
                               sring.at[slot], isem.at[slot])
              pltpu.async_copy(locF.at[r, s, pl.ds(g * G, G)],
                               lring.at[slot], isem.at[slot])

          @pl.when(g >= NGRP)
          def _():
              pltpu.async_copy(srcC.at[r, s, pl.ds((g - NGRP) * G, G)],
                               sring.at[slot], isem.at[slot])
              pltpu.async_copy(locC.at[r, s, pl.ds((g - NGRP) * G, G)],
                               lring.at[slot], isem.at[slot])

      pltpu.sync_copy(zeros_d, accum_sh.at[pl.ds(zrow, WPT)])

      @pl.when(s == 0)
      def _():
          pltpu.sync_copy(zeros_d.at[pl.ds(0, NPA - NROW)],
                          accum_sh.at[pl.ds(NROW, NPA - NROW)])

      stage(jnp.int32(0), jnp.int32(0))
      plsc.subcore_barrier()

      def grp(g, carry):
          slot = lax.rem(g, 2)
          pltpu.make_async_copy(srcA.at[0, 0, 0, pl.ds(0, G)],
                                sring.at[slot], isem.at[slot]).wait()
          pltpu.make_async_copy(srcA.at[0, 0, 0, pl.ds(0, G)],
                                lring.at[slot], isem.at[slot]).wait()

          @pl.when(g + 1 < ngroups)
          def _():
              stage(g + 1, 1 - slot)

          pltpu.async_copy(feat2e.at[sring.at[slot, 0]], rows_a, sem_a)

          def pair(u, c2):
              j0 = 2 * u
              pltpu.async_copy(feat2e.at[sring.at[slot, j0 + 1]], rows_b,
                               sem_b)
              pltpu.make_async_copy(feat2e.at[sring.at[slot, j0]], rows_a,
                                    sem_a).wait()
              pltpu.sync_copy(rows_a, accum_sh.at[lring.at[slot, j0]],
                              add=True)

              @pl.when(j0 + 2 < G)
              def _():
                  pltpu.async_copy(feat2e.at[sring.at[slot, j0 + 2]],
                                   rows_a, sem_a)

              pltpu.make_async_copy(feat2e.at[sring.at[slot, j0 + 1]],
                                    rows_b, sem_b).wait()
              pltpu.sync_copy(rows_b, accum_sh.at[lring.at[slot, j0 + 1]],
                              add=True)
              return c2

          lax.fori_loop(0, G // 2, pair, 0)
          return carry

      lax.fori_loop(0, ngroups, grp, 0)
      plsc.subcore_barrier()

      pltpu.sync_copy(accum_sh.at[pl.ds(zrow, WPT)],
                      agg_out.at[r, c, pl.ds(zrow, WPT)])

      @pl.when(jnp.logical_and(s == 0, c == r))
      def _():
          pltpu.sync_copy(accum_sh.at[pl.ds(NROW, NG)], cnt_out.at[r])

      plsc.subcore_barrier()

  return _sc_aggregate


BN = 400
NB = N // BN


def _ln_relu(x, g, b):
    mu = jnp.mean(x, axis=-1, keepdims=True)
    var = jnp.mean((x - mu) ** 2, axis=-1, keepdims=True)
    return jax.nn.relu((x - mu) / jnp.sqrt(var + 1e-5) * g + b)


def _tc_body(feat_ref, agg_ref, cnt_ref,
             W00, b00, W01, b01, g0, t0,
             W10, b10, W11, b11, g1, t1,
             ws1_ref, ws2_ref, out_ref):
    i = pl.program_id(0)
    feat_b = feat_ref[...]
    params = ((W00, b00, W01, b01, g0, t0), (W10, b10, W11, b11, g1, t1))
    hp = jax.lax.Precision.HIGHEST
    hs = []
    scores = []
    for r in range(R):
        agg_r = jnp.concatenate([agg_ref[r, 0], agg_ref[r, 1]], axis=-1)
        cnt_r = cnt_ref[r, :, 0:1]
        h = feat_b + agg_r / jnp.maximum(cnt_r, 1.0)
        Wa, ba, Wb, bb, g, b = params[r]
        h = _ln_relu(jnp.dot(h, Wa[...], preferred_element_type=jnp.float32,
                             precision=hp) + ba[...], g[...], b[...])
        h = _ln_relu(jnp.dot(h, Wb[...], preferred_element_type=jnp.float32,
                             precision=hp) + bb[...], g[...], b[...])
        t = jnp.tanh(jnp.dot(h, ws1_ref[r], preferred_element_type=jnp.float32,
                             precision=hp))
        scores.append(jnp.sum(t * ws2_ref[r], axis=-1, keepdims=True))
        hs.append(h)
    m = jnp.maximum(scores[0], scores[1])
    e0 = jnp.exp(scores[0] - m)
    e1 = jnp.exp(scores[1] - m)
    h_out = (e0 * hs[0] + e1 * hs[1]) / (e0 + e1)
    part = jnp.sum(h_out, axis=0, keepdims=True) * (1.0 / N)

    @pl.when(i == 0)
    def _():
        out_ref[...] = jnp.zeros_like(out_ref)

    out_ref[...] += part


def kernel(feat, edge_index, W0_0, b0_0, W0_1, b0_1, ln_g0, ln_b0,
           W1_0, b1_0, W1_1, b1_1, ln_g1, ln_b1, ws1, ws2):
    ei = edge_index.astype(jnp.int32)
    src = ei[:, 0, :]
    dst = ei[:, 1, :]
    pads = ((0, 0), (0, EP - E))
    srcA = jnp.pad(jnp.stack([src * 2, src * 2 + 1], axis=1),
                   ((0, 0),) + pads).reshape(R, NC, NS, NCH, K)
    srcC = jnp.pad(2 * N + (dst & 127), pads,
                   constant_values=2 * N).reshape(R, NS, NCH, K)
    locF = jnp.pad(dst, pads, constant_values=GR).reshape(R, NS, NCH, K)
    locC = jnp.pad(NROW + (dst >> 7), pads,
                   constant_values=GR).reshape(R, NS, NCH, K)
    feat2e = jnp.concatenate(
        [feat.reshape(2 * N, DH), jnp.eye(DH, dtype=jnp.float32)], axis=0)
    zeros_d = jnp.zeros((WPT, DH), jnp.float32)

    agg, cntg = _build_sc_aggregate()(srcA, srcC, locF, locC, feat2e,
                                      zeros_d)
    cnt = jnp.broadcast_to(cntg.reshape(R, NPC, 1), (R, NPC, CL))

    row = lambda v: v.reshape(1, D)
    out = pl.pallas_call(
        _tc_body,
        grid=(NB,),
        in_specs=[
            pl.BlockSpec((BN, D), lambda i: (i, 0)),
            pl.BlockSpec((R, NC, BN, DH), lambda i: (0, 0, i, 0)),
            pl.BlockSpec((R, BN, CL), lambda i: (0, i, 0)),
        ] + [
            spec
            for _ in range(R)
            for spec in (
                pl.BlockSpec((D, D), lambda i: (0, 0)),
                pl.BlockSpec((1, D), lambda i: (0, 0)),
                pl.BlockSpec((D, D), lambda i: (0, 0)),
                pl.BlockSpec((1, D), lambda i: (0, 0)),
                pl.BlockSpec((1, D), lambda i: (0, 0)),
                pl.BlockSpec((1, D), lambda i: (0, 0)),
            )
        ] + [
            pl.BlockSpec((R, D, DA), lambda i: (0, 0, 0)),
            pl.BlockSpec((R, 1, DA), lambda i: (0, 0, 0)),
        ],
        out_specs=pl.BlockSpec((1, D), lambda i: (0, 0)),
        out_shape=jax.ShapeDtypeStruct((1, D), jnp.float32),
        compiler_params=pltpu.CompilerParams(
            dimension_semantics=("arbitrary",)),
    )(feat, agg, cnt,
      W0_0, row(b0_0), W0_1, row(b0_1), row(ln_g0), row(ln_b0),
      W1_0, row(b1_0), W1_1, row(b1_1), row(ln_g1), row(ln_b1),
      ws1, ws2.reshape(R, 1, DA))
    return out


# speedup vs baseline: 1.3021x; 1.3021x over previous
import functools

import jax
import jax.numpy as jnp
from jax import lax
from jax.experimental import pallas as pl
from jax.experimental.pallas import tpu as pltpu
from jax.experimental.pallas import tpu_sc as plsc

N = 10000
E = 160000
R = 2
D = 256
DA = 16
DH = D // 2

NC = 2
NS = 16
K = 80
NCH = 128
G = 16
NGRP = NCH // G
EPC = NCH * K
EP = EPC * NS
NROW = 10112
NG = 80
NPA = NROW + NG + 8
GR = NPA - 1
NP = NROW
CL = 16
NPC = NG * 128
WPT = NROW // NS


@functools.lru_cache(maxsize=1)
def _build_sc_aggregate():
  mesh = plsc.VectorSubcoreMesh(core_axis_name="c", subcore_axis_name="s",
                                num_cores=NC, num_subcores=NS)

  @functools.partial(
      pl.kernel,
      out_type=(
          jax.ShapeDtypeStruct((R, NC, NP, DH), jnp.float32),
          jax.ShapeDtypeStruct((R, NG, 128), jnp.float32),
      ),
      mesh=mesh,
      scratch_types=(
          pltpu.VMEM((2, G, K), jnp.int32),
          pltpu.VMEM((2, G, K), jnp.int32),
          pltpu.VMEM((K, DH), jnp.float32),
          pltpu.VMEM((K, DH), jnp.float32),
          pltpu.VMEM_SHARED((NPA, DH), jnp.float32),
          pltpu.SemaphoreType.DMA,
          pltpu.SemaphoreType.DMA,
          pltpu.SemaphoreType.DMA((2,)),
      ),
  )
  def _sc_aggregate(srcA, srcC, locF, locC, feat2e, zeros_d,
                    agg_out, cnt_out,
                    sring, lring, rows_a, rows_b, accum_sh, sem_a, sem_b,
                    isem):
    c = lax.axis_index("c")
    s = lax.axis_index("s")
    zrow = s * WPT

    for r in range(R):
      ngroups = NGRP + jnp.where(c == r, NGRP, 0)

      def stage(g, slot):
          @pl.when(g < NGRP)
          def _():
              pltpu.async_copy(srcA.at[r, c, s, pl.ds(g * G, G)],
                               sring.at[slot], isem.at[slot])
              pltpu.async_copy(locF.at[r, s, pl.ds(g * G, G)],
                               lring.at[slot], isem.at[slot])

          @pl.when(g >= NGRP)
          def _():
              pltpu.async_copy(srcC.at[r, s, pl.ds((g - NGRP) * G, G)],
                               sring.at[slot], isem.at[slot])
              pltpu.async_copy(locC.at[r, s, pl.ds((g - NGRP) * G, G)],
                               lring.at[slot], isem.at[slot])

      pltpu.sync_copy(zeros_d, accum_sh.at[pl.ds(zrow, WPT)])

      @pl.when(s == 0)
      def _():
          pltpu.sync_copy(zeros_d.at[pl.ds(0, NPA - NROW)],
                          accum_sh.at[pl.ds(NROW, NPA - NROW)])

      stage(jnp.int32(0), jnp.int32(0))
      plsc.subcore_barrier()

      def grp(g, carry):
          slot = lax.rem(g, 2)
          pltpu.make_async_copy(srcA.at[0, 0, 0, pl.ds(0, G)],
                                sring.at[slot], isem.at[slot]).wait()
          pltpu.make_async_copy(srcA.at[0, 0, 0, pl.ds(0, G)],
                                lring.at[slot], isem.at[slot]).wait()

          @pl.when(g + 1 < ngroups)
          def _():
              stage(g + 1, 1 - slot)

          pltpu.async_copy(feat2e.at[sring.at[slot, 0]], rows_a, sem_a)

          def pair(u, c2):
              j0 = 2 * u
              pltpu.async_copy(feat2e.at[sring.at[slot, j0 + 1]], rows_b,
                               sem_b)
              pltpu.make_async_copy(feat2e.at[sring.at[slot, j0]], rows_a,
                                    sem_a).wait()
              pltpu.sync_copy(rows_a, accum_sh.at[lring.at[slot, j0]],
                              add=True)

              @pl.when(j0 + 2 < G)
              def _():
                  pltpu.async_copy(feat2e.at[sring.at[slot, j0 + 2]],
                                   rows_a, sem_a)

              pltpu.make_async_copy(feat2e.at[sring.at[slot, j0 + 1]],
                                    rows_b, sem_b).wait()
              pltpu.sync_copy(rows_b, accum_sh.at[lring.at[slot, j0 + 1]],
                              add=True)
              return c2

          lax.fori_loop(0, G // 2, pair, 0)
          return carry

      lax.fori_loop(0, ngroups, grp, 0)
      plsc.subcore_barrier()

      pltpu.sync_copy(accum_sh.at[pl.ds(zrow, WPT)],
                      agg_out.at[r, c, pl.ds(zrow, WPT)])

      @pl.when(jnp.logical_and(s == 0, c == r))
      def _():
          pltpu.sync_copy(accum_sh.at[pl.ds(NROW, NG)], cnt_out.at[r])

      plsc.subcore_barrier()

  return _sc_aggregate


BN = 400
NB = N // BN


def _ln_relu(x, g, b):
    mu = jnp.mean(x, axis=-1, keepdims=True)
    var = jnp.mean((x - mu) ** 2, axis=-1, keepdims=True)
    return jax.nn.relu((x - mu) / jnp.sqrt(var + 1e-5) * g + b)


def _tc_body(feat_ref, agg_ref, cnt_ref,
             W00, b00, W01, b01, g0, t0,
             W10, b10, W11, b11, g1, t1,
             ws1_ref, ws2_ref, out_ref):
    i = pl.program_id(0)
    feat_b = feat_ref[...]
    params = ((W00, b00, W01, b01, g0, t0), (W10, b10, W11, b11, g1, t1))
    hp = jax.lax.Precision.HIGHEST
    hs = []
    scores = []
    for r in range(R):
        agg_r = jnp.concatenate([agg_ref[r, 0], agg_ref[r, 1]], axis=-1)
        cnt_r = cnt_ref[r, :, 0:1]
        h = feat_b + agg_r / jnp.maximum(cnt_r, 1.0)
        Wa, ba, Wb, bb, g, b = params[r]
        h = _ln_relu(jnp.dot(h, Wa[...], preferred_element_type=jnp.float32,
                             precision=hp) + ba[...], g[...], b[...])
        h = _ln_relu(jnp.dot(h, Wb[...], preferred_element_type=jnp.float32,
                             precision=hp) + bb[...], g[...], b[...])
        t = jnp.tanh(jnp.dot(h, ws1_ref[r], preferred_element_type=jnp.float32,
                             precision=hp))
        scores.append(jnp.sum(t * ws2_ref[r], axis=-1, keepdims=True))
        hs.append(h)
    m = jnp.maximum(scores[0], scores[1])
    e0 = jnp.exp(scores[0] - m)
    e1 = jnp.exp(scores[1] - m)
    h_out = (e0 * hs[0] + e1 * hs[1]) / (e0 + e1)
    part = jnp.sum(h_out, axis=0, keepdims=True) * (1.0 / N)

    @pl.when(i == 0)
    def _():
        out_ref[...] = jnp.zeros_like(out_ref)

    out_ref[...] += part


def kernel(feat, edge_index, W0_0, b0_0, W0_1, b0_1, ln_g0, ln_b0,
           W1_0, b1_0, W1_1, b1_1, ln_g1, ln_b1, ws1, ws2):
    ei = edge_index.astype(jnp.int32)
    src = ei[:, 0, :]
    dst = ei[:, 1, :]
    pads = ((0, 0), (0, EP - E))
    srcA = jnp.pad(jnp.stack([src * 2, src * 2 + 1], axis=1),
                   ((0, 0),) + pads).reshape(R, NC, NS, NCH, K)
    srcC = jnp.pad(2 * N + (dst & 127), pads,
                   constant_values=2 * N).reshape(R, NS, NCH, K)
    locF = jnp.pad(dst, pads, constant_values=GR).reshape(R, NS, NCH, K)
    locC = jnp.pad(NROW + (dst >> 7), pads,
                   constant_values=GR).reshape(R, NS, NCH, K)
    feat2e = jnp.concatenate(
        [feat.reshape(2 * N, DH), jnp.eye(DH, dtype=jnp.float32)], axis=0)
    zeros_d = jnp.zeros((WPT, DH), jnp.float32)

    agg, cntg = _build_sc_aggregate()(srcA, srcC, locF, locC, feat2e,
                                      zeros_d)
    cnt = jnp.broadcast_to(cntg.reshape(R, NPC, 1), (R, NPC, CL))

    row = lambda v: v.reshape(1, D)
    out = pl.pallas_call(
        _tc_body,
        grid=(NB,),
        in_specs=[
            pl.BlockSpec((BN, D), lambda i: (i, 0)),
            pl.BlockSpec((R, NC, BN, DH), lambda i: (0, 0, i, 0)),
            pl.BlockSpec((R, BN, CL), lambda i: (0, i, 0)),
        ] + [
            spec
            for _ in range(R)
            for spec in (
                pl.BlockSpec((D, D), lambda i: (0, 0)),
                pl.BlockSpec((1, D), lambda i: (0, 0)),
                pl.BlockSpec((D, D), lambda i: (0, 0)),
                pl.BlockSpec((1, D), lambda i: (0, 0)),
                pl.BlockSpec((1, D), lambda i: (0, 0)),
                pl.BlockSpec((1, D), lambda i: (0, 0)),
            )
        ] + [
            pl.BlockSpec((R, D, DA), lambda i: (0, 0, 0)),
            pl.BlockSpec((R, 1, DA), lambda i: (0, 0, 0)),
        ],
        out_specs=pl.BlockSpec((1, D), lambda i: (0, 0)),
        out_shape=jax.ShapeDtypeStruct((1, D), jnp.float32),
        compiler_params=pltpu.CompilerParams(
            dimension_semantics=("arbitrary",)),
    )(feat, agg, cnt,
      W0_0, row(b0_0), W0_1, row(b0_1), row(ln_g0), row(ln_b0),
      W1_0, row(b1_0), W1_1, row(b1_1), row(ln_g1), row(ln_b1),
      ws1, ws2.reshape(R, 1, DA))
    return out


# speedup vs baseline: 1.3360x; 1.0260x over previous
import functools

import jax
import jax.numpy as jnp
from jax import lax
from jax.experimental import pallas as pl
from jax.experimental.pallas import tpu as pltpu
from jax.experimental.pallas import tpu_sc as plsc

N = 10000
E = 160000
R = 2
D = 256
DA = 16
DH = D // 2

NC = 2
NS = 16
K = 128
NCH = 80
G = 8
NGRP = NCH // G
EPC = NCH * K
EP = EPC * NS
NROW = 10112
NG = 80
NPA = NROW + NG + 8
GR = NPA - 1
NP = NROW
CL = 16
NPC = NG * 128
WPT = NROW // NS


@functools.lru_cache(maxsize=1)
def _build_sc_aggregate():
  mesh = plsc.VectorSubcoreMesh(core_axis_name="c", subcore_axis_name="s",
                                num_cores=NC, num_subcores=NS)

  @functools.partial(
      pl.kernel,
      out_type=(
          jax.ShapeDtypeStruct((R, NC, NP, DH), jnp.float32),
          jax.ShapeDtypeStruct((R, NG, 128), jnp.float32),
      ),
      mesh=mesh,
      scratch_types=(
          pltpu.VMEM((2, G, K), jnp.int32),
          pltpu.VMEM((2, G, K), jnp.int32),
          pltpu.VMEM((K, DH), jnp.float32),
          pltpu.VMEM((K, DH), jnp.float32),
          pltpu.VMEM_SHARED((NPA, DH), jnp.float32),
          pltpu.SemaphoreType.DMA,
          pltpu.SemaphoreType.DMA,
          pltpu.SemaphoreType.DMA((2,)),
      ),
  )
  def _sc_aggregate(srcA, srcC, locF, locC, feat2e, zeros_d,
                    agg_out, cnt_out,
                    sring, lring, rows_a, rows_b, accum_sh, sem_a, sem_b,
                    isem):
    c = lax.axis_index("c")
    s = lax.axis_index("s")
    zrow = s * WPT

    for r in range(R):
      ngroups = NGRP + jnp.where(c == r, NGRP, 0)

      def stage(g, slot):
          @pl.when(g < NGRP)
          def _():
              pltpu.async_copy(srcA.at[r, c, s, pl.ds(g * G, G)],
                               sring.at[slot], isem.at[slot])
              pltpu.async_copy(locF.at[r, s, pl.ds(g * G, G)],
                               lring.at[slot], isem.at[slot])

          @pl.when(g >= NGRP)
          def _():
              pltpu.async_copy(srcC.at[r, s, pl.ds((g - NGRP) * G, G)],
                               sring.at[slot], isem.at[slot])
              pltpu.async_copy(locC.at[r, s, pl.ds((g - NGRP) * G, G)],
                               lring.at[slot], isem.at[slot])

      pltpu.sync_copy(zeros_d, accum_sh.at[pl.ds(zrow, WPT)])

      @pl.when(s == 0)
      def _():
          pltpu.sync_copy(zeros_d.at[pl.ds(0, NPA - NROW)],
                          accum_sh.at[pl.ds(NROW, NPA - NROW)])

      stage(jnp.int32(0), jnp.int32(0))
      plsc.subcore_barrier()

      def grp(g, carry):
          slot = lax.rem(g, 2)
          pltpu.make_async_copy(srcA.at[0, 0, 0, pl.ds(0, G)],
                                sring.at[slot], isem.at[slot]).wait()
          pltpu.make_async_copy(srcA.at[0, 0, 0, pl.ds(0, G)],
                                lring.at[slot], isem.at[slot]).wait()

          @pl.when(g + 1 < ngroups)
          def _():
              stage(g + 1, 1 - slot)

          pltpu.async_copy(feat2e.at[sring.at[slot, 0]], rows_a, sem_a)

          def pair(u, c2):
              j0 = 2 * u
              pltpu.async_copy(feat2e.at[sring.at[slot, j0 + 1]], rows_b,
                               sem_b)
              pltpu.make_async_copy(feat2e.at[sring.at[slot, j0]], rows_a,
                                    sem_a).wait()
              pltpu.sync_copy(rows_a, accum_sh.at[lring.at[slot, j0]],
                              add=True)

              @pl.when(j0 + 2 < G)
              def _():
                  pltpu.async_copy(feat2e.at[sring.at[slot, j0 + 2]],
                                   rows_a, sem_a)

              pltpu.make_async_copy(feat2e.at[sring.at[slot, j0 + 1]],
                                    rows_b, sem_b).wait()
              pltpu.sync_copy(rows_b, accum_sh.at[lring.at[slot, j0 + 1]],
                              add=True)
              return c2

          lax.fori_loop(0, G // 2, pair, 0)
          return carry

      lax.fori_loop(0, ngroups, grp, 0)
      plsc.subcore_barrier()

      pltpu.sync_copy(accum_sh.at[pl.ds(zrow, WPT)],
                      agg_out.at[r, c, pl.ds(zrow, WPT)])

      @pl.when(jnp.logical_and(s == 0, c == r))
      def _():
          pltpu.sync_copy(accum_sh.at[pl.ds(NROW, NG)], cnt_out.at[r])

      plsc.subcore_barrier()

  return _sc_aggregate


BN = 400
NB = N // BN


def _ln_relu(x, g, b):
    mu = jnp.mean(x, axis=-1, keepdims=True)
    var = jnp.mean((x - mu) ** 2, axis=-1, keepdims=True)
    return jax.nn.relu((x - mu) / jnp.sqrt(var + 1e-5) * g + b)


def _tc_body(feat_ref, agg_ref, cnt_ref,
             W00, b00, W01, b01, g0, t0,
             W10, b10, W11, b11, g1, t1,
             ws1_ref, ws2_ref, out_ref):
    i = pl.program_id(0)
    feat_b = feat_ref[...]
    params = ((W00, b00, W01, b01, g0, t0), (W10, b10, W11, b11, g1, t1))
    hp = jax.lax.Precision.HIGHEST
    hs = []
    scores = []
    for r in range(R):
        agg_r = jnp.concatenate([agg_ref[r, 0], agg_ref[r, 1]], axis=-1)
        cnt_r = cnt_ref[r, :, 0:1]
        h = feat_b + agg_r / jnp.maximum(cnt_r, 1.0)
        Wa, ba, Wb, bb, g, b = params[r]
        h = _ln_relu(jnp.dot(h, Wa[...], preferred_element_type=jnp.float32,
                             precision=hp) + ba[...], g[...], b[...])
        h = _ln_relu(jnp.dot(h, Wb[...], preferred_element_type=jnp.float32,
                             precision=hp) + bb[...], g[...], b[...])
        t = jnp.tanh(jnp.dot(h, ws1_ref[r], preferred_element_type=jnp.float32,
                             precision=hp))
        scores.append(jnp.sum(t * ws2_ref[r], axis=-1, keepdims=True))
        hs.append(h)
    m = jnp.maximum(scores[0], scores[1])
    e0 = jnp.exp(scores[0] - m)
    e1 = jnp.exp(scores[1] - m)
    h_out = (e0 * hs[0] + e1 * hs[1]) / (e0 + e1)
    part = jnp.sum(h_out, axis=0, keepdims=True) * (1.0 / N)

    @pl.when(i == 0)
    def _():
        out_ref[...] = jnp.zeros_like(out_ref)

    out_ref[...] += part


def kernel(feat, edge_index, W0_0, b0_0, W0_1, b0_1, ln_g0, ln_b0,
           W1_0, b1_0, W1_1, b1_1, ln_g1, ln_b1, ws1, ws2):
    ei = edge_index.astype(jnp.int32)
    src = ei[:, 0, :]
    dst = ei[:, 1, :]
    pads = ((0, 0), (0, EP - E))
    srcA = jnp.pad(jnp.stack([src * 2, src * 2 + 1], axis=1),
                   ((0, 0),) + pads).reshape(R, NC, NS, NCH, K)
    srcC = jnp.pad(2 * N + (dst & 127), pads,
                   constant_values=2 * N).reshape(R, NS, NCH, K)
    locF = jnp.pad(dst, pads, constant_values=GR).reshape(R, NS, NCH, K)
    locC = jnp.pad(NROW + (dst >> 7), pads,
                   constant_values=GR).reshape(R, NS, NCH, K)
    feat2e = jnp.concatenate(
        [feat.reshape(2 * N, DH), jnp.eye(DH, dtype=jnp.float32)], axis=0)
    zeros_d = jnp.zeros((WPT, DH), jnp.float32)

    agg, cntg = _build_sc_aggregate()(srcA, srcC, locF, locC, feat2e,
                                      zeros_d)
    cnt = jnp.broadcast_to(cntg.reshape(R, NPC, 1), (R, NPC, CL))

    row = lambda v: v.reshape(1, D)
    out = pl.pallas_call(
        _tc_body,
        grid=(NB,),
        in_specs=[
            pl.BlockSpec((BN, D), lambda i: (i, 0)),
            pl.BlockSpec((R, NC, BN, DH), lambda i: (0, 0, i, 0)),
            pl.BlockSpec((R, BN, CL), lambda i: (0, i, 0)),
        ] + [
            spec
            for _ in range(R)
            for spec in (
                pl.BlockSpec((D, D), lambda i: (0, 0)),
                pl.BlockSpec((1, D), lambda i: (0, 0)),
                pl.BlockSpec((D, D), lambda i: (0, 0)),
                pl.BlockSpec((1, D), lambda i: (0, 0)),
                pl.BlockSpec((1, D), lambda i: (0, 0)),
                pl.BlockSpec((1, D), lambda i: (0, 0)),
            )
        ] + [
            pl.BlockSpec((R, D, DA), lambda i: (0, 0, 0)),
            pl.BlockSpec((R, 1, DA), lambda i: (0, 0, 0)),
        ],
        out_specs=pl.BlockSpec((1, D), lambda i: (0, 0)),
        out_shape=jax.ShapeDtypeStruct((1, D), jnp.float32),
        compiler_params=pltpu.CompilerParams(
            dimension_semantics=("arbitrary",)),
    )(feat, agg, cnt,
      W0_0, row(b0_0), W0_1, row(b0_1), row(ln_g0), row(ln_b0),
      W1_0, row(b1_0), W1_1, row(b1_1), row(ln_g1), row(ln_b1),
      ws1, ws2.reshape(R, 1, DA))
    return out


# speedup vs baseline: 1.3430x; 1.0052x over previous
import functools

import jax
import jax.numpy as jnp
from jax import lax
from jax.experimental import pallas as pl
from jax.experimental.pallas import tpu as pltpu
from jax.experimental.pallas import tpu_sc as plsc

N = 10000
E = 160000
R = 2
D = 256
DA = 16
DH = D // 2

NC = 2
NS = 16
K = 80
NCH = 126
EPC = NCH * K
EP = EPC * NS
NPH = 5056
NG = 80
NPA = NPH + NG + 8
GR = NPA - 1
NP = 2 * NPH
CL = 16
NPC = NG * 128
WPT = NPH // 8


@functools.lru_cache(maxsize=1)
def _build_sc_aggregate():
  mesh = plsc.VectorSubcoreMesh(core_axis_name="c", subcore_axis_name="s",
                                num_cores=NC, num_subcores=NS)

  @functools.partial(
      pl.kernel,
      out_type=(
          jax.ShapeDtypeStruct((R, NC, NP, DH), jnp.float32),
          jax.ShapeDtypeStruct((R, NG, 128), jnp.float32),
      ),
      mesh=mesh,
      scratch_types=(
          pltpu.VMEM((2 * NCH, K), jnp.int32),
          pltpu.VMEM((2 * NCH, K), jnp.int32),
          pltpu.VMEM((K, DH), jnp.float32),
          pltpu.VMEM((K, DH), jnp.float32),
          pltpu.VMEM_SHARED((NPA, DH), jnp.float32),
          pltpu.SemaphoreType.DMA,
          pltpu.SemaphoreType.DMA,
      ),
  )
  def _sc_aggregate(srcA, srcC, locP, locC, feat2e, zeros_d,
                    agg_out, cnt_out,
                    src_v, loc_v, rows_a, rows_b, accum_sh, sem_a, sem_b):
    c = lax.axis_index("c")
    s = lax.axis_index("s")
    zrow = jnp.minimum(s, 7) * WPT
    for r in range(R):
      pltpu.sync_copy(srcA.at[r, c, s], src_v.at[pl.ds(0, NCH)])
      pltpu.sync_copy(srcC.at[r, s], src_v.at[pl.ds(NCH, NCH)])
      for p in range(2):
        pltpu.sync_copy(locP.at[r, p, s], loc_v.at[pl.ds(0, NCH)])
        if p == 0:
            pltpu.sync_copy(locC.at[r, s], loc_v.at[pl.ds(NCH, NCH)])

        @pl.when(s < 8)
        def _():
            pltpu.sync_copy(zeros_d, accum_sh.at[pl.ds(zrow, WPT)])

        @pl.when(s == 8)
        def _():
            pltpu.sync_copy(zeros_d.at[pl.ds(0, NPA - NPH)],
                            accum_sh.at[pl.ds(NPH, NPA - NPH)])

        plsc.subcore_barrier()

        if p == 0:
            nch = NCH + jnp.where(c == r, NCH, 0)
        else:
            nch = NCH

        pltpu.async_copy(feat2e.at[src_v.at[0]], rows_a, sem_a)

        def pair(t, carry):
            j0 = 2 * t
            pltpu.async_copy(feat2e.at[src_v.at[j0 + 1]], rows_b, sem_b)
            pltpu.make_async_copy(feat2e.at[src_v.at[j0]], rows_a,
                                  sem_a).wait()
            pltpu.sync_copy(rows_a, accum_sh.at[loc_v.at[j0]], add=True)

            @pl.when(j0 + 2 < nch)
            def _():
                pltpu.async_copy(feat2e.at[src_v.at[j0 + 2]], rows_a, sem_a)

            pltpu.make_async_copy(feat2e.at[src_v.at[j0 + 1]], rows_b,
                                  sem_b).wait()
            pltpu.sync_copy(rows_b, accum_sh.at[loc_v.at[j0 + 1]], add=True)
            return carry

        lax.fori_loop(0, nch // 2, pair, 0)
        plsc.subcore_barrier()

        @pl.when(s < 8)
        def _():
            pltpu.sync_copy(accum_sh.at[pl.ds(zrow, WPT)],
                            agg_out.at[r, c, pl.ds(p * NPH + zrow, WPT)])

        if p == 0:
          @pl.when(jnp.logical_and(s == 8, c == r))
          def _():
              pltpu.sync_copy(accum_sh.at[pl.ds(NPH, NG)], cnt_out.at[r])

        plsc.subcore_barrier()

  return _sc_aggregate


BN = 400
NB = N // BN


def _ln_relu(x, g, b):
    mu = jnp.mean(x, axis=-1, keepdims=True)
    var = jnp.mean((x - mu) ** 2, axis=-1, keepdims=True)
    return jax.nn.relu((x - mu) / jnp.sqrt(var + 1e-5) * g + b)


def _tc_body(feat_ref, agg_ref, cnt_ref,
             W00, b00, W01, b01, g0, t0,
             W10, b10, W11, b11, g1, t1,
             ws1_ref, ws2_ref, out_ref):
    i = pl.program_id(0)
    feat_b = feat_ref[...]
    params = ((W00, b00, W01, b01, g0, t0), (W10, b10, W11, b11, g1, t1))
    hp = jax.lax.Precision.HIGHEST
    hs = []
    scores = []
    for r in range(R):
        agg_r = jnp.concatenate([agg_ref[r, 0], agg_ref[r, 1]], axis=-1)
        cnt_r = cnt_ref[r, :, 0:1]
        h = feat_b + agg_r / jnp.maximum(cnt_r, 1.0)
        Wa, ba, Wb, bb, g, b = params[r]
        h = _ln_relu(jnp.dot(h, Wa[...], preferred_element_type=jnp.float32,
                             precision=hp) + ba[...], g[...], b[...])
        h = _ln_relu(jnp.dot(h, Wb[...], preferred_element_type=jnp.float32,
                             precision=hp) + bb[...], g[...], b[...])
        t = jnp.tanh(jnp.dot(h, ws1_ref[r], preferred_element_type=jnp.float32,
                             precision=hp))
        scores.append(jnp.sum(t * ws2_ref[r], axis=-1, keepdims=True))
        hs.append(h)
    m = jnp.maximum(scores[0], scores[1])
    e0 = jnp.exp(scores[0] - m)
    e1 = jnp.exp(scores[1] - m)
    h_out = (e0 * hs[0] + e1 * hs[1]) / (e0 + e1)
    part = jnp.sum(h_out, axis=0, keepdims=True) * (1.0 / N)

    @pl.when(i == 0)
    def _():
        out_ref[...] = jnp.zeros_like(out_ref)

    out_ref[...] += part


def kernel(feat, edge_index, W0_0, b0_0, W0_1, b0_1, ln_g0, ln_b0,
           W1_0, b1_0, W1_1, b1_1, ln_g1, ln_b1, ws1, ws2):
    ei = edge_index.astype(jnp.int32)
    src = ei[:, 0, :]
    dst = ei[:, 1, :]
    pads = ((0, 0), (0, EP - E))
    srcA = jnp.pad(jnp.stack([src * 2, src * 2 + 1], axis=1),
                   ((0, 0),) + pads).reshape(R, NC, NS, NCH, K)
    srcC = jnp.pad(2 * N + (dst & 127), pads,
                   constant_values=2 * N).reshape(R, NS, NCH, K)
    loc0 = jnp.where(dst < NPH, dst, GR)
    loc1 = jnp.where(dst >= NPH, dst - NPH, GR)
    locP = jnp.pad(jnp.stack([loc0, loc1], axis=1), ((0, 0),) + pads,
                   constant_values=GR).reshape(R, 2, NS, NCH, K)
    locC = jnp.pad(NPH + (dst >> 7), pads,
                   constant_values=GR).reshape(R, NS, NCH, K)
    feat2e = jnp.concatenate(
        [feat.reshape(2 * N, DH), jnp.eye(DH, dtype=jnp.float32)], axis=0)
    zeros_d = jnp.zeros((WPT, DH), jnp.float32)

    agg, cntg = _build_sc_aggregate()(srcA, srcC, locP, locC, feat2e,
                                      zeros_d)
    cnt = jnp.broadcast_to(cntg.reshape(R, NPC, 1), (R, NPC, CL))

    row = lambda v: v.reshape(1, D)
    out = pl.pallas_call(
        _tc_body,
        grid=(NB,),
        in_specs=[
            pl.BlockSpec((BN, D), lambda i: (i, 0)),
            pl.BlockSpec((R, NC, BN, DH), lambda i: (0, 0, i, 0)),
            pl.BlockSpec((R, BN, CL), lambda i: (0, i, 0)),
        ] + [
            spec
            for _ in range(R)
            for spec in (
                pl.BlockSpec((D, D), lambda i: (0, 0)),
                pl.BlockSpec((1, D), lambda i: (0, 0)),
                pl.BlockSpec((D, D), lambda i: (0, 0)),
                pl.BlockSpec((1, D), lambda i: (0, 0)),
                pl.BlockSpec((1, D), lambda i: (0, 0)),
                pl.BlockSpec((1, D), lambda i: (0, 0)),
            )
        ] + [
            pl.BlockSpec((R, D, DA), lambda i: (0, 0, 0)),
            pl.BlockSpec((R, 1, DA), lambda i: (0, 0, 0)),
        ],
        out_specs=pl.BlockSpec((1, D), lambda i: (0, 0)),
        out_shape=jax.ShapeDtypeStruct((1, D), jnp.float32),
        compiler_params=pltpu.CompilerParams(
            dimension_semantics=("arbitrary",)),
    )(feat, agg, cnt,
      W0_0, row(b0_0), W0_1, row(b0_1), row(ln_g0), row(ln_b0),
      W1_0, row(b1_0), W1_1, row(b1_1), row(ln_g1), row(ln_b1),
      ws1, ws2.reshape(R, 1, DA))
    return out
